# baseline (speedup 1.0000x reference)
import jax
import jax.numpy as jnp
from jax import lax
from jax.experimental import pallas as pl

_N = 10000
_THR = 0.01


def _epilogue_body(src_ref, dst_ref, inner_ref, aa_ref, bb_ref, wsum_ref, out_ref):
    src = src_ref[...]
    dst = dst_ref[...]
    inner = inner_ref[...]
    denom = jnp.sqrt(aa_ref[...]) * jnp.sqrt(bb_ref[...]) + 1e-08
    keep = (inner / denom) >= _THR
    v = jnp.where(keep, wsum_ref[...], 0.0)
    out_ref[...] = jnp.where(src == dst, 2.0 * v, v)


def kernel(edge_index, edge_weight, features):
    E = edge_index.shape[1]
    src = edge_index[0]
    dst = edge_index[1]
    lin = src * _N + dst

    lin_s, w_s = lax.sort((lin, edge_weight), num_keys=1)
    csum = jnp.cumsum(w_s)
    pfx = jnp.concatenate([jnp.zeros((1,), jnp.float32), csum])
    q = jnp.minimum(src, dst) * _N + jnp.maximum(src, dst)
    lo = jnp.searchsorted(lin_s, q, side="left")
    hi = jnp.searchsorted(lin_s, q, side="right")
    wsum = pfx[hi] - pfx[lo]

    fs = features[src]
    fd = features[dst]
    inner = jnp.sum(fs * fd, axis=-1)
    aa = jnp.sum(fs * fs, axis=-1)
    bb = jnp.sum(fd * fd, axis=-1)

    shape2d = (E // 128, 128)
    out = pl.pallas_call(
        _epilogue_body,
        out_shape=jax.ShapeDtypeStruct(shape2d, jnp.float32),
    )(
        src.reshape(shape2d),
        dst.reshape(shape2d),
        inner.reshape(shape2d),
        aa.reshape(shape2d),
        bb.reshape(shape2d),
        wsum.reshape(shape2d),
    )
    return out.reshape(E)


# speedup vs baseline: 4.4780x; 4.4780x over previous
import functools

import jax
import jax.numpy as jnp
from jax import lax
from jax.experimental import pallas as pl
from jax.experimental.pallas import tpu as pltpu
from jax.experimental.pallas import tpu_sc as plsc

_N = 10000
_THR = 0.01

_EP = 323584
_BT = _EP // 16
_BC = _BT // 128
_QT = _EP // 32
_QC = _QT // 128
_GARB = 2 ** 19
_TBL = 2 ** 19 + 16

_HASH_LEVELS = (
    (2654435761, 13, 2 ** 19),
    (2246822519, 13, 2 ** 19),
    (3266489917, 15, 2 ** 17),
    (668265263, 17, 2 ** 15),
    (374761393, 17, 2 ** 15),
)


def _join_body(src_h, dst_h, w_h, out_h, kbuf, wbuf, qbuf, acc, idxrow,
               win, vst, qk, qv, dstage, neg1, zf, tkeys, tvals):
    s = lax.axis_index("s")
    c = lax.axis_index("c")
    wid = s * 2 + c
    bbase = s * _BT
    qbase = wid * _QT

    pltpu.sync_copy(src_h.at[pl.ds(bbase, _BT)], kbuf)
    pltpu.sync_copy(w_h.at[pl.ds(bbase, _BT)], wbuf)

    def mk_keys(i, carry):
        pltpu.sync_copy(dst_h.at[pl.ds(bbase + i * 256, 256)], dstage)
        for j in range(16):
            sl = pl.ds(i * 256 + j * 16, 16)
            kbuf[sl] = kbuf[sl] * _N + dstage[pl.ds(j * 16, 16)]
        return carry

    lax.fori_loop(0, _BT // 256, mk_keys, 0)

    def init_acc(i, carry):
        acc[pl.ds(i * 16, 16)] = jnp.zeros((16,), jnp.float32)
        return carry

    lax.fori_loop(0, _QT // 16, init_acc, 0)

    def init_consts(i, carry):
        neg1[pl.ds(i * 16, 16)] = jnp.full((16,), -1, jnp.int32)
        zf[pl.ds(i * 16, 16)] = jnp.zeros((16,), jnp.float32)
        return carry

    lax.fori_loop(0, 64, init_consts, 0)

    pltpu.sync_copy(src_h.at[pl.ds(qbase, _QT)], qbuf)

    def mk_queries(i, carry):
        pltpu.sync_copy(dst_h.at[pl.ds(qbase + i * 128, 128)],
                        dstage.at[pl.ds(0, 128)])
        for j in range(8):
            sl = pl.ds(i * 128 + j * 16, 16)
            s16 = qbuf[sl]
            d16 = dstage[pl.ds(j * 16, 16)]
            qbuf[sl] = (jnp.minimum(s16, d16) * _N + jnp.maximum(s16, d16))
        return carry

    lax.fori_loop(0, _QC, mk_queries, 0)

    for cst, sh, hrange in _HASH_LEVELS:
        ucst = jnp.uint32(cst)
        ush = jnp.uint32(sh)
        stripe = hrange // 16
        sbase = s * stripe

        for off in range(0, stripe, 1024):
            clen = min(1024, stripe - off)
            pltpu.sync_copy(neg1.at[pl.ds(0, clen)],
                            tkeys.at[pl.ds(sbase + off, clen)])
            pltpu.sync_copy(zf.at[pl.ds(0, clen)],
                            tvals.at[pl.ds(sbase + off, clen)])
        plsc.subcore_barrier()

        def hash_row(buf, i):
            for j in range(8):
                k16 = buf[pl.ds(i * 128 + j * 16, 16)]
                h = (k16.astype(jnp.uint32) * ucst) >> ush
                idxrow[0, pl.ds(j * 16, 16)] = jnp.where(
                    k16 < 0, jnp.int32(_GARB), h.astype(jnp.int32))

        def bscat(i, carry):
            hash_row(kbuf, i)
            pltpu.sync_copy(kbuf.at[pl.ds(i * 128, 128)],
                            tkeys.at[idxrow.at[0]])
            return carry

        lax.fori_loop(0, _BC, bscat, 0)
        plsc.subcore_barrier()

        def bfin(i, carry):
            hash_row(kbuf, i)
            pltpu.sync_copy(tkeys.at[idxrow.at[0]], win)
            for j in range(8):
                sl = pl.ds(i * 128 + j * 16, 16)
                j16 = pl.ds(j * 16, 16)
                k16 = kbuf[sl]
                won = (win[j16] == k16) & (k16 >= 0)
                vst[j16] = jnp.where(won, wbuf[sl], 0.0)
                kbuf[sl] = jnp.where(won, -5, k16)
            pltpu.sync_copy(vst, tvals.at[idxrow.at[0]], add=True)
            return carry

        lax.fori_loop(0, _BC, bfin, 0)
        plsc.subcore_barrier()

        def qprobe(i, carry):
            hash_row(qbuf, i)
            pltpu.sync_copy(tkeys.at[idxrow.at[0]], qk)
            pltpu.sync_copy(tvals.at[idxrow.at[0]], qv)
            for j in range(8):
                sl = pl.ds(i * 128 + j * 16, 16)
                j16 = pl.ds(j * 16, 16)
                found = qk[j16] == qbuf[sl]
                acc[sl] = acc[sl] + jnp.where(found, qv[j16], 0.0)
            return carry

        lax.fori_loop(0, _QC, qprobe, 0)
        plsc.subcore_barrier()

    pltpu.sync_copy(acc, out_h.at[pl.ds(qbase, _QT)])


@jax.jit
def _sc_join(src_p, dst_p, w_p):
    mesh = plsc.VectorSubcoreMesh(core_axis_name="c", subcore_axis_name="s")
    return pl.kernel(
        _join_body,
        out_type=jax.ShapeDtypeStruct((_EP,), jnp.float32),
        mesh=mesh,
        scratch_types=[
            pltpu.VMEM((_BT,), jnp.int32),
            pltpu.VMEM((_BT,), jnp.float32),
            pltpu.VMEM((_QT,), jnp.int32),
            pltpu.VMEM((_QT,), jnp.float32),
            pltpu.VMEM((1, 128), jnp.int32),
            pltpu.VMEM((128,), jnp.int32),
            pltpu.VMEM((128,), jnp.float32),
            pltpu.VMEM((128,), jnp.int32),
            pltpu.VMEM((128,), jnp.float32),
            pltpu.VMEM((256,), jnp.int32),
            pltpu.VMEM((1024,), jnp.int32),
            pltpu.VMEM((1024,), jnp.float32),
            pltpu.VMEM_SHARED((_TBL,), jnp.int32),
            pltpu.VMEM_SHARED((_TBL,), jnp.float32),
        ],
    )(src_p, dst_p, w_p)


def _epilogue_body(src_ref, dst_ref, inner_ref, aa_ref, bb_ref, wsum_ref,
                   out_ref):
    src = src_ref[...]
    dst = dst_ref[...]
    inner = inner_ref[...]
    denom = jnp.sqrt(aa_ref[...]) * jnp.sqrt(bb_ref[...]) + 1e-08
    keep = (inner / denom) >= _THR
    v = jnp.where(keep, wsum_ref[...], 0.0)
    out_ref[...] = jnp.where(src == dst, 2.0 * v, v)


def kernel(edge_index, edge_weight, features):
    E = edge_index.shape[1]
    src = edge_index[0]
    dst = edge_index[1]

    pad = _EP - E
    src_p = jnp.concatenate([src, jnp.full((pad,), _N, jnp.int32)])
    dst_p = jnp.concatenate([dst, jnp.zeros((pad,), jnp.int32)])
    w_p = jnp.concatenate([edge_weight, jnp.zeros((pad,), jnp.float32)])

    wsum = _sc_join(src_p, dst_p, w_p)[:E]

    fs = features[src]
    fd = features[dst]
    inner = jnp.sum(fs * fd, axis=-1)
    aa = jnp.sum(fs * fs, axis=-1)
    bb = jnp.sum(fd * fd, axis=-1)

    shape2d = (E // 128, 128)
    out = pl.pallas_call(
        _epilogue_body,
        out_shape=jax.ShapeDtypeStruct(shape2d, jnp.float32),
    )(
        src.reshape(shape2d),
        dst.reshape(shape2d),
        inner.reshape(shape2d),
        aa.reshape(shape2d),
        bb.reshape(shape2d),
        wsum.reshape(shape2d),
    )
    return out.reshape(E)
